# baseline (speedup 1.0000x reference)
import functools

import jax
import jax.numpy as jnp
import numpy as np
from jax.experimental import pallas as pl
from jax.experimental.pallas import tpu as pltpu
from jax.experimental.pallas import tpu_sc as plsc

_HIDDEN = 128
_HEADS = 8
_HD = _HIDDEN // _HEADS
_NL = 2
_RELS = [
    ("regulates", "cpg", "gene"),
    ("regulated_by", "gene", "cpg"),
    ("targets", "mirna", "gene"),
    ("targeted_by", "gene", "mirna"),
    ("interacts", "gene", "gene"),
    ("self_gene", "gene", "gene"),
    ("self_cpg", "cpg", "cpg"),
    ("self_mirna", "mirna", "mirna"),
]
_NT = ["gene", "cpg", "mirna"]
_SRC_RELS = {t: [n for (n, st, dt) in _RELS if st == t] for t in _NT}
_DST_RELS = {t: [n for (n, st, dt) in _RELS if dt == t] for t in _NT}

_SC_CORES = 2
_SC_TILES = 16
_NW = _SC_CORES * _SC_TILES
_GCHUNK = 256


def _sc_gather(table, idx):
    M = idx.shape[0]
    assert M % (_NW * _GCHUNK) == 0
    per_w = M // _NW
    n_chunks = per_w // _GCHUNK
    mesh = plsc.VectorSubcoreMesh(core_axis_name="c", subcore_axis_name="s")

    @functools.partial(
        pl.kernel,
        mesh=mesh,
        out_type=jax.ShapeDtypeStruct((M, _HIDDEN), jnp.float32),
        scratch_types=[
            pltpu.VMEM((_GCHUNK,), jnp.int32),
            pltpu.VMEM((_GCHUNK, _HIDDEN), jnp.float32),
            pltpu.SemaphoreType.DMA,
        ],
    )
    def k(table_hbm, idx_hbm, out_hbm, idx_v, rows_v, sem):
        wid = jax.lax.axis_index("s") * _SC_CORES + jax.lax.axis_index("c")
        base = wid * per_w

        @pl.loop(0, n_chunks)
        def _(ci):
            off = base + ci * _GCHUNK
            pltpu.sync_copy(idx_hbm.at[pl.ds(off, _GCHUNK)], idx_v)
            pltpu.async_copy(table_hbm.at[idx_v], rows_v, sem).wait()
            pltpu.sync_copy(rows_v, out_hbm.at[pl.ds(off, _GCHUNK)])

    return k(table, idx)


def _build_gather_indices(edge_index, n_nodes, req):
    gidx = {}
    slices = {}
    for t in _NT:
        parts = []
        pos = 0
        for name in _SRC_RELS[t]:
            s = _SRC_RELS[t].index(name)
            src = edge_index[name][0]
            parts.append(src + s * n_nodes[t])
            slices.setdefault(name, {})["src"] = pos
            pos += src.shape[0]
        for name in _DST_RELS[t]:
            s = len(_SRC_RELS[t]) + _DST_RELS[t].index(name)
            dst = edge_index[name][1]
            parts.append(dst + s * n_nodes[t])
            slices.setdefault(name, {})["dst"] = pos
            pos += dst.shape[0]
        cat = jnp.concatenate(parts)
        need = max(pos, req.get(t, 0))
        tot = _round_up(need, _NW * _GCHUNK)
        if tot > pos:
            cat = jnp.concatenate([cat, jnp.zeros((tot - pos,), jnp.int32)])
        gidx[t] = cat
    return gidx, slices


_RS = 32640
_ACC_ROWS = _RS + 128
_SC_C = 512
_EPAD = 8192


def _round_up(v, m):
    return (v + m - 1) // m * m


def _sc_scatter_layer(plan, P_all, dstp_list):
    mesh = plsc.VectorSubcoreMesh(core_axis_name="c", subcore_axis_name="s")
    rounds = []
    for ri, rel in enumerate(plan):
        for g in range(len(rel["ranges"])):
            for r9 in range(9):
                rounds.append((ri, g, r9))
    loads = [0, 0]
    per_core = ([], [])
    for rnd in sorted(rounds, key=lambda z: -plan[z[0]]["ne16"]):
        c = 0 if loads[0] <= loads[1] else 1
        per_core[c].append(rnd)
        loads[c] += plan[rnd[0]]["ne16"]

    n_p = len(P_all)
    n_d = len(dstp_list)
    out_types = [jax.ShapeDtypeStruct((9 * rel["ndf"], _HD), jnp.float32)
                 for rel in plan]

    @functools.partial(
        pl.kernel,
        mesh=mesh,
        out_type=out_types,
        compiler_params=pltpu.CompilerParams(use_tc_tiling_on_sc=False),
        scratch_types=[
            pltpu.VMEM((_SC_C,), jnp.int32),
            pltpu.VMEM((_SC_C, _HD), jnp.float32),
            pltpu.VMEM((512, _HD), jnp.float32),
            pltpu.VMEM_SHARED((_ACC_ROWS, _HD), jnp.float32),
        ],
    )
    def k(*refs):
        ps = refs[:n_p]
        ds = refs[n_p:n_p + n_d]
        outs = refs[n_p + n_d:n_p + n_d + len(plan)]
        idx_v, pay_v, zbuf, acc = refs[n_p + n_d + len(plan):]
        cid = jax.lax.axis_index("c")
        tid = jax.lax.axis_index("s")

        @pl.loop(0, 512)
        def _(i):
            zbuf[pl.ds(i, 1), pl.ds(0, 16)] = jnp.zeros((1, 16), jnp.float32)

        for c in (0, 1):
            for (ri, g, r9) in per_core[c]:
                @pl.when(cid == c)
                def _round(ri=ri, g=g, r9=r9):
                    rel = plan[ri]
                    lo, fr = rel["ranges"][g]
                    zrows = (fr + 128) // 16
                    zn, zt = zrows // 512, zrows % 512
                    zbase = tid * zrows

                    @pl.loop(0, zn)
                    def _(zi):
                        pltpu.sync_copy(zbuf,
                                        acc.at[pl.ds(zbase + zi * 512, 512)])
                    if zt:
                        pltpu.sync_copy(zbuf.at[pl.ds(0, zt)],
                                        acc.at[pl.ds(zbase + zn * 512, zt)])
                    plsc.subcore_barrier()
                    ne16 = rel["ne16"]
                    base = tid * ne16
                    p_ref = ps[ri * 9 + r9]
                    d_ref = ds[rel["dstp_base"] + g]

                    @pl.loop(0, ne16 // _SC_C)
                    def _(ci):
                        off = base + ci * _SC_C
                        pltpu.sync_copy(d_ref.at[pl.ds(off, _SC_C)], idx_v)
                        pltpu.sync_copy(p_ref.at[pl.ds(off, _SC_C)], pay_v)
                        pltpu.sync_copy(pay_v, acc.at[idx_v], add=True)
                    plsc.subcore_barrier()
                    frt = fr // 16
                    pltpu.sync_copy(
                        acc.at[pl.ds(tid * frt, frt)],
                        outs[ri].at[pl.ds(r9 * rel["ndf"] + lo + tid * frt, frt)])
                    plsc.subcore_barrier()

    return k(*P_all, *dstp_list)


def _proj_body(x_ref, w_ref, b_ref, o_ref, *, k):
    y = (
        jnp.dot(x_ref[...], w_ref[...], preferred_element_type=jnp.float32, precision=jax.lax.Precision.HIGHEST)
        + b_ref[...]
    )
    for s in range(k):
        o_ref[s, :, :] = y[:, s * _HIDDEN:(s + 1) * _HIDDEN]


def _proj(x, W, b, blk=1000):
    N = x.shape[0]
    K = W.shape[1]
    k = K // _HIDDEN
    return pl.pallas_call(
        functools.partial(_proj_body, k=k),
        grid=(pl.cdiv(N, blk),),
        in_specs=[
            pl.BlockSpec((blk, _HIDDEN), lambda i: (i, 0)),
            pl.BlockSpec((_HIDDEN, K), lambda i: (0, 0)),
            pl.BlockSpec((1, K), lambda i: (0, 0)),
        ],
        out_specs=pl.BlockSpec((k, blk, _HIDDEN), lambda i: (0, i, 0)),
        out_shape=jax.ShapeDtypeStruct((k, N, _HIDDEN), jnp.float32),
    )(x, W, b.reshape(1, K))


def _edge_body(xj_ref, xi_ref, attbd_ref, exp_ref, *o_refs):
    xj = xj_ref[...]
    xi = xi_ref[...]
    s = xi + xj
    e = jnp.where(s > 0, s, 0.2 * s)
    alpha = jnp.dot(e, attbd_ref[...], preferred_element_type=jnp.float32, precision=jax.lax.Precision.HIGHEST)
    ex = jnp.exp(alpha)
    w = xj * jnp.dot(ex, exp_ref[...], preferred_element_type=jnp.float32, precision=jax.lax.Precision.HIGHEST)
    for r in range(_HEADS):
        o_refs[r][...] = w[:, _HD * r:_HD * (r + 1)]
    o_refs[_HEADS][...] = jnp.concatenate(
        [ex, jnp.zeros((ex.shape[0], _HD - _HEADS), jnp.float32)], axis=1)


_EBLK = 1000


def _edge_compute(G_src, G_dst, pos_s, pos_d, ne_pad, att):
    nb = pl.cdiv(ne_pad, _EBLK)
    assert pos_s % _EBLK == 0 and pos_d % _EBLK == 0
    pbs = pos_s // _EBLK
    pbd = pos_d // _EBLK
    attbd = att.reshape(-1)[:, None] * jnp.repeat(
        jnp.eye(_HEADS, dtype=jnp.float32), _HD, axis=0)
    expand = jnp.repeat(jnp.eye(_HEADS, dtype=jnp.float32), _HD, axis=1)
    return pl.pallas_call(
        _edge_body,
        grid=(nb,),
        in_specs=[
            pl.BlockSpec((_EBLK, _HIDDEN), lambda i: (pbs + i, 0)),
            pl.BlockSpec((_EBLK, _HIDDEN), lambda i: (pbd + i, 0)),
            pl.BlockSpec((_HIDDEN, _HEADS), lambda i: (0, 0)),
            pl.BlockSpec((_HEADS, _HIDDEN), lambda i: (0, 0)),
        ],
        out_specs=[pl.BlockSpec((_EBLK, _HD), lambda i: (i, 0))] * 9,
        out_shape=[jax.ShapeDtypeStruct((ne_pad, _HD), jnp.float32)] * 9,
    )(G_src, G_dst, attbd, expand)


def _final_body(x_ref, gam_ref, bet_ref, bias_ref, exp_ref, *o_refs):
    out_ref = o_refs[-1]
    agg = None
    for ri, o_ref in enumerate(o_refs[:-1]):
        ob = o_ref[...]
        num = jnp.concatenate([ob[r] for r in range(_HEADS)], axis=1)
        den = ob[_HEADS][:, 0:_HEADS]
        den_rep = jnp.dot(den, exp_ref[...],
                          preferred_element_type=jnp.float32, precision=jax.lax.Precision.HIGHEST) + 1e-16
        o = num / den_rep + bias_ref[ri:ri + 1, :]
        agg = o if agg is None else agg + o
    h = jnp.where(agg > 0, agg, jnp.exp(agg) - 1.0)
    y = x_ref[...] + h
    mu = jnp.mean(y, axis=-1, keepdims=True)
    var = jnp.mean((y - mu) ** 2, axis=-1, keepdims=True)
    out_ref[...] = (y - mu) / jnp.sqrt(var + 1e-5) * gam_ref[...] + bet_ref[...]


def _finalize(x, gamma, beta, biases, Os):
    N = x.shape[0]
    R = len(Os)
    blk = _EBLK
    expand = jnp.repeat(jnp.eye(_HEADS, dtype=jnp.float32), _HD, axis=1)
    return pl.pallas_call(
        _final_body,
        grid=(N // blk,),
        in_specs=[
            pl.BlockSpec((blk, _HIDDEN), lambda i: (i, 0)),
            pl.BlockSpec((1, _HIDDEN), lambda i: (0, 0)),
            pl.BlockSpec((1, _HIDDEN), lambda i: (0, 0)),
            pl.BlockSpec((R, _HIDDEN), lambda i: (0, 0)),
            pl.BlockSpec((_HEADS, _HIDDEN), lambda i: (0, 0)),
        ] + [pl.BlockSpec((9, blk, _HD), lambda i: (0, i, 0))] * R,
        out_specs=pl.BlockSpec((blk, _HIDDEN), lambda i: (i, 0)),
        out_shape=jax.ShapeDtypeStruct((N, _HIDDEN), jnp.float32),
    )(x, gamma.reshape(1, -1), beta.reshape(1, -1), jnp.stack(biases), expand,
      *Os)


def _head_body(b_ref, x_ref, gam_ref, bet_ref, o_ref, *, scale):
    y = jnp.dot(b_ref[...], x_ref[...],
                preferred_element_type=jnp.float32, precision=jax.lax.Precision.HIGHEST) * scale
    mu = jnp.mean(y, axis=-1, keepdims=True)
    var = jnp.mean((y - mu) ** 2, axis=-1, keepdims=True)
    o_ref[...] = (y - mu) / jnp.sqrt(var + 1e-5) * gam_ref[...] + bet_ref[...]


def _head(batch, x, gamma, beta):
    B, F = batch.shape
    return pl.pallas_call(
        functools.partial(_head_body, scale=1.0 / np.sqrt(F)),
        out_shape=jax.ShapeDtypeStruct((B, _HIDDEN), jnp.float32),
    )(batch, x, gamma.reshape(1, -1), beta.reshape(1, -1))


def kernel(batch_gene, batch_meth, batch_mirna, edge_index, params):
    n_nodes = {t: params["node_emb"][t].shape[0] for t in _NT}

    plan = []
    dstp_list = []
    for ri, (name, st, dt) in enumerate(_RELS):
        dst = edge_index[name][1]
        ne = dst.shape[0]
        ne_pad = _round_up(ne, _EPAD)
        ndf = _round_up(n_nodes[dt], 128)
        ranges = []
        lo = 0
        while lo < ndf:
            fr = min(_RS, ndf - lo)
            ranges.append((lo, fr))
            lo += fr
        dstp_base = len(dstp_list)
        for (lo, fr) in ranges:
            ok = (dst >= lo) & (dst < lo + fr)
            arr = jnp.where(ok, dst - lo, fr).astype(jnp.int32)
            dstp_list.append(jnp.concatenate(
                [arr, jnp.full((ne_pad - ne,), fr, jnp.int32)]))
        plan.append(dict(name=name, ne_pad=ne_pad, ne16=ne_pad // 16,
                         ndf=ndf, ranges=ranges, dstp_base=dstp_base))

    req = {t: 0 for t in _NT}
    pos_probe = {}
    for t in _NT:
        pos = 0
        for name in _SRC_RELS[t]:
            pos_probe[(name, "src")] = pos
            pos += edge_index[name][0].shape[0]
        for name in _DST_RELS[t]:
            pos_probe[(name, "dst")] = pos
            pos += edge_index[name][1].shape[0]
    for ri, (name, st, dt) in enumerate(_RELS):
        nb = -(-plan[ri]["ne_pad"] // _EBLK) * _EBLK
        req[st] = max(req[st], pos_probe[(name, "src")] + nb)
        req[dt] = max(req[dt], pos_probe[(name, "dst")] + nb)

    gidx, gslices = _build_gather_indices(edge_index, n_nodes, req)

    x = {t: params["node_emb"][t] for t in _NT}

    for l in range(_NL):
        conv = params["convs"][l]
        table = {}
        for t in _NT:
            Ws = [conv[n]["Wl"] for n in _SRC_RELS[t]] + [conv[n]["Wr"] for n in _DST_RELS[t]]
            bs = [conv[n]["bl"] for n in _SRC_RELS[t]] + [conv[n]["br"] for n in _DST_RELS[t]]
            table[t] = _proj(x[t], jnp.concatenate(Ws, axis=1), jnp.concatenate(bs, axis=0))

        G = {t: _sc_gather(table[t].reshape(-1, _HIDDEN), gidx[t]) for t in _NT}

        P_all = []
        for ri, (name, st, dt) in enumerate(_RELS):
            P = _edge_compute(G[st], G[dt], gslices[name]["src"],
                              gslices[name]["dst"], plan[ri]["ne_pad"],
                              conv[name]["att"])
            P_all.extend(P)
        Oraw = _sc_scatter_layer(plan, P_all, dstp_list)
        O = {name: Oraw[ri].reshape(9, plan[ri]["ndf"], _HD)
             for ri, (name, _, _) in enumerate(_RELS)}

        nxt = {}
        for t in _NT:
            Os = [O[n] for n in _DST_RELS[t]]
            biases = [conv[n]["bias"] for n in _DST_RELS[t]]
            ln = params["norms"][l][t]
            nxt[t] = _finalize(x[t], ln["gamma"], ln["beta"], biases, Os)
        x = nxt

    on = params["out_norm"]
    z_gene = _head(batch_gene, x["gene"], on["gene"]["gamma"], on["gene"]["beta"])
    z_cpg = _head(batch_meth, x["cpg"], on["cpg"]["gamma"], on["cpg"]["beta"])
    z_mirna = _head(batch_mirna, x["mirna"], on["mirna"]["gamma"], on["mirna"]["beta"])
    return (z_gene, z_cpg, z_mirna)


# speedup vs baseline: 1.5033x; 1.5033x over previous
import functools

import jax
import jax.numpy as jnp
import numpy as np
from jax.experimental import pallas as pl
from jax.experimental.pallas import tpu as pltpu
from jax.experimental.pallas import tpu_sc as plsc

_HIDDEN = 128
_HEADS = 8
_HD = _HIDDEN // _HEADS
_NL = 2
_RELS = [
    ("regulates", "cpg", "gene"),
    ("regulated_by", "gene", "cpg"),
    ("targets", "mirna", "gene"),
    ("targeted_by", "gene", "mirna"),
    ("interacts", "gene", "gene"),
    ("self_gene", "gene", "gene"),
    ("self_cpg", "cpg", "cpg"),
    ("self_mirna", "mirna", "mirna"),
]
_NT = ["gene", "cpg", "mirna"]
_SRC_RELS = {t: [n for (n, st, dt) in _RELS if st == t] for t in _NT}
_DST_RELS = {t: [n for (n, st, dt) in _RELS if dt == t] for t in _NT}

_SC_CORES = 2
_SC_TILES = 16
_NW = _SC_CORES * _SC_TILES
_GCHUNK = 96


def _sc_gather(table, idx):
    M = idx.shape[0]
    assert M % (_NW * _GCHUNK) == 0
    per_w = M // _NW
    n_chunks = per_w // _GCHUNK
    mesh = plsc.VectorSubcoreMesh(core_axis_name="c", subcore_axis_name="s")

    @functools.partial(
        pl.kernel,
        mesh=mesh,
        out_type=jax.ShapeDtypeStruct((M, _HIDDEN), jnp.float32),
        scratch_types=[
            pltpu.VMEM((_GCHUNK,), jnp.int32),
            pltpu.VMEM((_GCHUNK,), jnp.int32),
            pltpu.VMEM((_GCHUNK, _HIDDEN), jnp.float32),
            pltpu.VMEM((_GCHUNK, _HIDDEN), jnp.float32),
            pltpu.SemaphoreType.DMA,
            pltpu.SemaphoreType.DMA,
            pltpu.SemaphoreType.DMA,
            pltpu.SemaphoreType.DMA,
        ],
    )
    def k(table_hbm, idx_hbm, out_hbm, ia, ib, ra, rb, si0, si1, sg0, sg1):
        wid = jax.lax.axis_index("s") * _SC_CORES + jax.lax.axis_index("c")
        base = wid * per_w

        @pl.loop(0, n_chunks // 2)
        def _(g):
            o0 = base + (2 * g) * _GCHUNK
            o1 = o0 + _GCHUNK
            h_i0 = pltpu.async_copy(idx_hbm.at[pl.ds(o0, _GCHUNK)], ia, si0)
            h_i1 = pltpu.async_copy(idx_hbm.at[pl.ds(o1, _GCHUNK)], ib, si1)
            h_i0.wait()
            h_g0 = pltpu.async_copy(table_hbm.at[ia], ra, sg0)
            h_i1.wait()
            h_g1 = pltpu.async_copy(table_hbm.at[ib], rb, sg1)
            h_g0.wait()
            pltpu.sync_copy(ra, out_hbm.at[pl.ds(o0, _GCHUNK)])
            h_g1.wait()
            pltpu.sync_copy(rb, out_hbm.at[pl.ds(o1, _GCHUNK)])

    return k(table, idx)


def _build_gather_indices(edge_index, n_nodes, req):
    gidx = {}
    slices = {}
    for t in _NT:
        parts = []
        pos = 0
        for name in _SRC_RELS[t]:
            s = _SRC_RELS[t].index(name)
            src = edge_index[name][0]
            parts.append(src + s * n_nodes[t])
            slices.setdefault(name, {})["src"] = pos
            pos += src.shape[0]
        for name in _DST_RELS[t]:
            s = len(_SRC_RELS[t]) + _DST_RELS[t].index(name)
            dst = edge_index[name][1]
            parts.append(dst + s * n_nodes[t])
            slices.setdefault(name, {})["dst"] = pos
            pos += dst.shape[0]
        cat = jnp.concatenate(parts)
        need = max(pos, req.get(t, 0))
        tot = _round_up(need, _NW * _GCHUNK * 2)
        if tot > pos:
            cat = jnp.concatenate([cat, jnp.zeros((tot - pos,), jnp.int32)])
        gidx[t] = cat
    return gidx, slices


_RS = 24448
_ACC_ROWS = _RS + 128
_SC_C = 256
_PW = 48
_EPAD = 4096


def _round_up(v, m):
    return (v + m - 1) // m * m


def _sc_scatter_layer(plan, P_all, dstp_list):
    mesh = plsc.VectorSubcoreMesh(core_axis_name="c", subcore_axis_name="s")
    rounds = []
    for ri, rel in enumerate(plan):
        for g in range(len(rel["ranges"])):
            for r9 in range(3):
                rounds.append((ri, g, r9))
    loads = [0, 0]
    per_core = ([], [])
    for rnd in sorted(rounds, key=lambda z: -plan[z[0]]["ne16"]):
        c = 0 if loads[0] <= loads[1] else 1
        per_core[c].append(rnd)
        loads[c] += plan[rnd[0]]["ne16"]

    n_p = len(P_all)
    n_d = len(dstp_list)
    out_types = [jax.ShapeDtypeStruct((3 * rel["ndf"], _PW), jnp.float32)
                 for rel in plan]

    @functools.partial(
        pl.kernel,
        mesh=mesh,
        out_type=out_types,
        compiler_params=pltpu.CompilerParams(use_tc_tiling_on_sc=False),
        scratch_types=[
            pltpu.VMEM((_SC_C,), jnp.int32),
            pltpu.VMEM((_SC_C,), jnp.int32),
            pltpu.VMEM((_SC_C, _PW), jnp.float32),
            pltpu.VMEM((_SC_C, _PW), jnp.float32),
            pltpu.VMEM_SHARED((_ACC_ROWS, _PW), jnp.float32),
            pltpu.SemaphoreType.DMA,
            pltpu.SemaphoreType.DMA,
            pltpu.SemaphoreType.DMA,
            pltpu.SemaphoreType.DMA,
        ],
    )
    def k(*refs):
        zz = refs[0]
        ps = refs[1:1 + n_p]
        ds = refs[1 + n_p:1 + n_p + n_d]
        outs = refs[1 + n_p + n_d:1 + n_p + n_d + len(plan)]
        (ixa, ixb, pya, pyb, acc,
         si0, si1, sp0, sp1) = refs[1 + n_p + n_d + len(plan):]
        cid = jax.lax.axis_index("c")
        tid = jax.lax.axis_index("s")

        for c in (0, 1):
            for (ri, g, r9) in per_core[c]:
                @pl.when(cid == c)
                def _round(ri=ri, g=g, r9=r9):
                    rel = plan[ri]
                    lo, fr = rel["ranges"][g]
                    zrows = (fr + 128) // 16
                    zbase = tid * zrows
                    pltpu.sync_copy(zz.at[pl.ds(0, zrows)],
                                    acc.at[pl.ds(zbase, zrows)])
                    plsc.subcore_barrier()
                    ne16 = rel["ne16"]
                    base = tid * ne16
                    p_ref = ps[ri * 3 + r9]
                    d_ref = ds[rel["dstp_base"] + g]
                    nc = ne16 // _SC_C

                    @pl.loop(0, nc // 2)
                    def _(ci):
                        o0 = base + (2 * ci) * _SC_C
                        o1 = o0 + _SC_C
                        h_i0 = pltpu.async_copy(d_ref.at[pl.ds(o0, _SC_C)],
                                                ixa, si0)
                        h_p0 = pltpu.async_copy(p_ref.at[pl.ds(o0, _SC_C)],
                                                pya, sp0)
                        h_i1 = pltpu.async_copy(d_ref.at[pl.ds(o1, _SC_C)],
                                                ixb, si1)
                        h_p1 = pltpu.async_copy(p_ref.at[pl.ds(o1, _SC_C)],
                                                pyb, sp1)
                        h_i0.wait()
                        h_p0.wait()
                        pltpu.sync_copy(pya, acc.at[ixa], add=True)
                        h_i1.wait()
                        h_p1.wait()
                        pltpu.sync_copy(pyb, acc.at[ixb], add=True)
                    if nc % 2:
                        ot = base + (nc - 1) * _SC_C
                        h_it = pltpu.async_copy(d_ref.at[pl.ds(ot, _SC_C)],
                                                ixa, si0)
                        h_pt = pltpu.async_copy(p_ref.at[pl.ds(ot, _SC_C)],
                                                pya, sp0)
                        h_it.wait()
                        h_pt.wait()
                        pltpu.sync_copy(pya, acc.at[ixa], add=True)
                    plsc.subcore_barrier()
                    frt = fr // 16
                    pltpu.sync_copy(
                        acc.at[pl.ds(tid * frt, frt)],
                        outs[ri].at[pl.ds(r9 * rel["ndf"] + lo + tid * frt, frt)])
                    plsc.subcore_barrier()

    zmax = max((fr + 128) // 16 for rel in plan for (_, fr) in rel["ranges"])
    zeros = jnp.zeros((zmax, _PW), jnp.float32)
    return k(zeros, *P_all, *dstp_list)


def _proj_body(x_ref, w_ref, b_ref, o_ref, *, k):
    y = (
        jnp.dot(x_ref[...], w_ref[...], preferred_element_type=jnp.float32, precision=jax.lax.Precision.HIGHEST)
        + b_ref[...]
    )
    for s in range(k):
        o_ref[s, :, :] = y[:, s * _HIDDEN:(s + 1) * _HIDDEN]


def _proj(x, W, b, blk=1000):
    N = x.shape[0]
    K = W.shape[1]
    k = K // _HIDDEN
    return pl.pallas_call(
        functools.partial(_proj_body, k=k),
        grid=(pl.cdiv(N, blk),),
        in_specs=[
            pl.BlockSpec((blk, _HIDDEN), lambda i: (i, 0)),
            pl.BlockSpec((_HIDDEN, K), lambda i: (0, 0)),
            pl.BlockSpec((1, K), lambda i: (0, 0)),
        ],
        out_specs=pl.BlockSpec((k, blk, _HIDDEN), lambda i: (0, i, 0)),
        out_shape=jax.ShapeDtypeStruct((k, N, _HIDDEN), jnp.float32),
    )(x, W, b.reshape(1, K))


def _edge_body(xj_ref, xi_ref, attbd_ref, exp_ref, *o_refs):
    xj = xj_ref[...]
    xi = xi_ref[...]
    s = xi + xj
    e = jnp.where(s > 0, s, 0.2 * s)
    alpha = jnp.dot(e, attbd_ref[...], preferred_element_type=jnp.float32, precision=jax.lax.Precision.HIGHEST)
    ex = jnp.exp(alpha)
    w = xj * jnp.dot(ex, exp_ref[...], preferred_element_type=jnp.float32, precision=jax.lax.Precision.HIGHEST)
    o_refs[0][...] = w[:, 0:48]
    o_refs[1][...] = w[:, 48:96]
    o_refs[2][...] = jnp.concatenate(
        [w[:, 96:128], ex, jnp.zeros((ex.shape[0], 8), jnp.float32)], axis=1)


_EBLK = 1000


def _edge_compute(G_src, G_dst, pos_s, pos_d, ne_pad, att):
    nb = pl.cdiv(ne_pad, _EBLK)
    assert pos_s % _EBLK == 0 and pos_d % _EBLK == 0
    pbs = pos_s // _EBLK
    pbd = pos_d // _EBLK
    attbd = att.reshape(-1)[:, None] * jnp.repeat(
        jnp.eye(_HEADS, dtype=jnp.float32), _HD, axis=0)
    expand = jnp.repeat(jnp.eye(_HEADS, dtype=jnp.float32), _HD, axis=1)
    return pl.pallas_call(
        _edge_body,
        grid=(nb,),
        in_specs=[
            pl.BlockSpec((_EBLK, _HIDDEN), lambda i: (pbs + i, 0)),
            pl.BlockSpec((_EBLK, _HIDDEN), lambda i: (pbd + i, 0)),
            pl.BlockSpec((_HIDDEN, _HEADS), lambda i: (0, 0)),
            pl.BlockSpec((_HEADS, _HIDDEN), lambda i: (0, 0)),
        ],
        out_specs=[pl.BlockSpec((_EBLK, _PW), lambda i: (i, 0))] * 3,
        out_shape=[jax.ShapeDtypeStruct((ne_pad, _PW), jnp.float32)] * 3,
    )(G_src, G_dst, attbd, expand)


def _final_body(x_ref, gam_ref, bet_ref, bias_ref, exp_ref, *o_refs):
    out_ref = o_refs[-1]
    agg = None
    for ri, o_ref in enumerate(o_refs[:-1]):
        ob = o_ref[...]
        num = jnp.concatenate([ob[0], ob[1], ob[2][:, 0:32]], axis=1)
        den = ob[2][:, 32:40]
        den_rep = jnp.dot(den, exp_ref[...],
                          preferred_element_type=jnp.float32, precision=jax.lax.Precision.HIGHEST) + 1e-16
        o = num / den_rep + bias_ref[ri:ri + 1, :]
        agg = o if agg is None else agg + o
    h = jnp.where(agg > 0, agg, jnp.exp(agg) - 1.0)
    y = x_ref[...] + h
    mu = jnp.mean(y, axis=-1, keepdims=True)
    var = jnp.mean((y - mu) ** 2, axis=-1, keepdims=True)
    out_ref[...] = (y - mu) / jnp.sqrt(var + 1e-5) * gam_ref[...] + bet_ref[...]


def _finalize(x, gamma, beta, biases, Os):
    N = x.shape[0]
    R = len(Os)
    blk = _EBLK
    expand = jnp.repeat(jnp.eye(_HEADS, dtype=jnp.float32), _HD, axis=1)
    return pl.pallas_call(
        _final_body,
        grid=(N // blk,),
        in_specs=[
            pl.BlockSpec((blk, _HIDDEN), lambda i: (i, 0)),
            pl.BlockSpec((1, _HIDDEN), lambda i: (0, 0)),
            pl.BlockSpec((1, _HIDDEN), lambda i: (0, 0)),
            pl.BlockSpec((R, _HIDDEN), lambda i: (0, 0)),
            pl.BlockSpec((_HEADS, _HIDDEN), lambda i: (0, 0)),
        ] + [pl.BlockSpec((3, blk, _PW), lambda i: (0, i, 0))] * R,
        out_specs=pl.BlockSpec((blk, _HIDDEN), lambda i: (i, 0)),
        out_shape=jax.ShapeDtypeStruct((N, _HIDDEN), jnp.float32),
    )(x, gamma.reshape(1, -1), beta.reshape(1, -1), jnp.stack(biases), expand,
      *Os)


def _head_body(b_ref, x_ref, gam_ref, bet_ref, o_ref, *, scale):
    y = jnp.dot(b_ref[...], x_ref[...],
                preferred_element_type=jnp.float32, precision=jax.lax.Precision.HIGHEST) * scale
    mu = jnp.mean(y, axis=-1, keepdims=True)
    var = jnp.mean((y - mu) ** 2, axis=-1, keepdims=True)
    o_ref[...] = (y - mu) / jnp.sqrt(var + 1e-5) * gam_ref[...] + bet_ref[...]


def _head(batch, x, gamma, beta):
    B, F = batch.shape
    return pl.pallas_call(
        functools.partial(_head_body, scale=1.0 / np.sqrt(F)),
        out_shape=jax.ShapeDtypeStruct((B, _HIDDEN), jnp.float32),
    )(batch, x, gamma.reshape(1, -1), beta.reshape(1, -1))


def kernel(batch_gene, batch_meth, batch_mirna, edge_index, params):
    n_nodes = {t: params["node_emb"][t].shape[0] for t in _NT}

    plan = []
    dstp_list = []
    for ri, (name, st, dt) in enumerate(_RELS):
        dst = edge_index[name][1]
        ne = dst.shape[0]
        ne_pad = _round_up(ne, _EPAD)
        ndf = _round_up(n_nodes[dt], 128)
        ranges = []
        lo = 0
        while lo < ndf:
            fr = min(_RS, ndf - lo)
            ranges.append((lo, fr))
            lo += fr
        dstp_base = len(dstp_list)
        for (lo, fr) in ranges:
            ok = (dst >= lo) & (dst < lo + fr)
            arr = jnp.where(ok, dst - lo, fr).astype(jnp.int32)
            dstp_list.append(jnp.concatenate(
                [arr, jnp.full((ne_pad - ne,), fr, jnp.int32)]))
        plan.append(dict(name=name, ne_pad=ne_pad, ne16=ne_pad // 16,
                         ndf=ndf, ranges=ranges, dstp_base=dstp_base))

    req = {t: 0 for t in _NT}
    pos_probe = {}
    for t in _NT:
        pos = 0
        for name in _SRC_RELS[t]:
            pos_probe[(name, "src")] = pos
            pos += edge_index[name][0].shape[0]
        for name in _DST_RELS[t]:
            pos_probe[(name, "dst")] = pos
            pos += edge_index[name][1].shape[0]
    for ri, (name, st, dt) in enumerate(_RELS):
        nb = -(-plan[ri]["ne_pad"] // _EBLK) * _EBLK
        req[st] = max(req[st], pos_probe[(name, "src")] + nb)
        req[dt] = max(req[dt], pos_probe[(name, "dst")] + nb)

    gidx, gslices = _build_gather_indices(edge_index, n_nodes, req)

    x = {t: params["node_emb"][t] for t in _NT}

    for l in range(_NL):
        conv = params["convs"][l]
        table = {}
        for t in _NT:
            Ws = [conv[n]["Wl"] for n in _SRC_RELS[t]] + [conv[n]["Wr"] for n in _DST_RELS[t]]
            bs = [conv[n]["bl"] for n in _SRC_RELS[t]] + [conv[n]["br"] for n in _DST_RELS[t]]
            table[t] = _proj(x[t], jnp.concatenate(Ws, axis=1), jnp.concatenate(bs, axis=0))

        G = {t: _sc_gather(table[t].reshape(-1, _HIDDEN), gidx[t]) for t in _NT}

        P_all = []
        for ri, (name, st, dt) in enumerate(_RELS):
            P = _edge_compute(G[st], G[dt], gslices[name]["src"],
                              gslices[name]["dst"], plan[ri]["ne_pad"],
                              conv[name]["att"])
            P_all.extend(P)
        Oraw = _sc_scatter_layer(plan, P_all, dstp_list)
        O = {name: Oraw[ri].reshape(3, plan[ri]["ndf"], _PW)
             for ri, (name, _, _) in enumerate(_RELS)}

        nxt = {}
        for t in _NT:
            Os = [O[n] for n in _DST_RELS[t]]
            biases = [conv[n]["bias"] for n in _DST_RELS[t]]
            ln = params["norms"][l][t]
            nxt[t] = _finalize(x[t], ln["gamma"], ln["beta"], biases, Os)
        x = nxt

    on = params["out_norm"]
    z_gene = _head(batch_gene, x["gene"], on["gene"]["gamma"], on["gene"]["beta"])
    z_cpg = _head(batch_meth, x["cpg"], on["cpg"]["gamma"], on["cpg"]["beta"])
    z_mirna = _head(batch_mirna, x["mirna"], on["mirna"]["gamma"], on["mirna"]["beta"])
    return (z_gene, z_cpg, z_mirna)


# speedup vs baseline: 1.5201x; 1.0112x over previous
import functools

import jax
import jax.numpy as jnp
import numpy as np
from jax.experimental import pallas as pl
from jax.experimental.pallas import tpu as pltpu
from jax.experimental.pallas import tpu_sc as plsc

_HIDDEN = 128
_HEADS = 8
_HD = _HIDDEN // _HEADS
_NL = 2
_RELS = [
    ("regulates", "cpg", "gene"),
    ("regulated_by", "gene", "cpg"),
    ("targets", "mirna", "gene"),
    ("targeted_by", "gene", "mirna"),
    ("interacts", "gene", "gene"),
    ("self_gene", "gene", "gene"),
    ("self_cpg", "cpg", "cpg"),
    ("self_mirna", "mirna", "mirna"),
]
_NT = ["gene", "cpg", "mirna"]
_SRC_RELS = {t: [n for (n, st, dt) in _RELS if st == t] for t in _NT}
_DST_RELS = {t: [n for (n, st, dt) in _RELS if dt == t] for t in _NT}

_SC_CORES = 2
_SC_TILES = 16
_NW = _SC_CORES * _SC_TILES
_GCHUNK = 96


def _sc_gather(table, idx):
    M = idx.shape[0]
    assert M % (_NW * _GCHUNK) == 0
    per_w = M // _NW
    n_chunks = per_w // _GCHUNK
    mesh = plsc.VectorSubcoreMesh(core_axis_name="c", subcore_axis_name="s")

    @functools.partial(
        pl.kernel,
        mesh=mesh,
        out_type=jax.ShapeDtypeStruct((M, _HIDDEN), jnp.float32),
        scratch_types=[
            pltpu.VMEM((_GCHUNK,), jnp.int32),
            pltpu.VMEM((_GCHUNK,), jnp.int32),
            pltpu.VMEM((_GCHUNK, _HIDDEN), jnp.float32),
            pltpu.VMEM((_GCHUNK, _HIDDEN), jnp.float32),
            pltpu.SemaphoreType.DMA,
            pltpu.SemaphoreType.DMA,
            pltpu.SemaphoreType.DMA,
            pltpu.SemaphoreType.DMA,
        ],
    )
    def k(table_hbm, idx_hbm, out_hbm, ia, ib, ra, rb, si0, si1, sg0, sg1):
        wid = jax.lax.axis_index("s") * _SC_CORES + jax.lax.axis_index("c")
        base = wid * per_w

        @pl.loop(0, n_chunks // 2)
        def _(g):
            o0 = base + (2 * g) * _GCHUNK
            o1 = o0 + _GCHUNK
            h_i0 = pltpu.async_copy(idx_hbm.at[pl.ds(o0, _GCHUNK)], ia, si0)
            h_i1 = pltpu.async_copy(idx_hbm.at[pl.ds(o1, _GCHUNK)], ib, si1)
            h_i0.wait()
            h_g0 = pltpu.async_copy(table_hbm.at[ia], ra, sg0)
            h_i1.wait()
            h_g1 = pltpu.async_copy(table_hbm.at[ib], rb, sg1)
            h_g0.wait()
            pltpu.sync_copy(ra, out_hbm.at[pl.ds(o0, _GCHUNK)])
            h_g1.wait()
            pltpu.sync_copy(rb, out_hbm.at[pl.ds(o1, _GCHUNK)])

    return k(table, idx)


def _build_gather_indices(edge_index, n_nodes, req):
    gidx = {}
    slices = {}
    for t in _NT:
        parts = []
        pos = 0
        for name in _SRC_RELS[t]:
            s = _SRC_RELS[t].index(name)
            src = edge_index[name][0]
            parts.append(src + s * n_nodes[t])
            slices.setdefault(name, {})["src"] = pos
            pos += src.shape[0]
        for name in _DST_RELS[t]:
            s = len(_SRC_RELS[t]) + _DST_RELS[t].index(name)
            dst = edge_index[name][1]
            parts.append(dst + s * n_nodes[t])
            slices.setdefault(name, {})["dst"] = pos
            pos += dst.shape[0]
        cat = jnp.concatenate(parts)
        need = max(pos, req.get(t, 0))
        tot = _round_up(need, _NW * _GCHUNK * 2)
        if tot > pos:
            cat = jnp.concatenate([cat, jnp.zeros((tot - pos,), jnp.int32)])
        gidx[t] = cat
    return gidx, slices


_RS = 24448
_ACC_ROWS = _RS + 128
_SC_C = 256
_PW = 48
_EPAD = 4096


def _round_up(v, m):
    return (v + m - 1) // m * m


def _sc_scatter_layer(plan, P_all, dstp_list):
    mesh = plsc.VectorSubcoreMesh(core_axis_name="c", subcore_axis_name="s")
    rounds = []
    for ri, rel in enumerate(plan):
        for g in range(len(rel["ranges"])):
            for r9 in range(3):
                rounds.append((ri, g, r9))
    def _cost(z):
        rel = plan[z[0]]
        fr = rel["ranges"][z[1]][1]
        return rel["ne16"] * 196 + 2 * ((fr + 128) // 16) * 192
    loads = [0, 0]
    per_core = ([], [])
    for rnd in sorted(rounds, key=lambda z: -_cost(z)):
        c = 0 if loads[0] <= loads[1] else 1
        per_core[c].append(rnd)
        loads[c] += _cost(rnd)

    n_p = len(P_all)
    n_d = len(dstp_list)
    out_types = [jax.ShapeDtypeStruct((3 * rel["ndf"], _PW), jnp.float32)
                 for rel in plan]

    @functools.partial(
        pl.kernel,
        mesh=mesh,
        out_type=out_types,
        compiler_params=pltpu.CompilerParams(use_tc_tiling_on_sc=False),
        scratch_types=[
            pltpu.VMEM((_SC_C,), jnp.int32),
            pltpu.VMEM((_SC_C,), jnp.int32),
            pltpu.VMEM((_SC_C, _PW), jnp.float32),
            pltpu.VMEM((_SC_C, _PW), jnp.float32),
            pltpu.VMEM_SHARED((_ACC_ROWS, _PW), jnp.float32),
            pltpu.SemaphoreType.DMA,
            pltpu.SemaphoreType.DMA,
            pltpu.SemaphoreType.DMA,
            pltpu.SemaphoreType.DMA,
        ],
    )
    def k(*refs):
        zz = refs[0]
        ps = refs[1:1 + n_p]
        ds = refs[1 + n_p:1 + n_p + n_d]
        outs = refs[1 + n_p + n_d:1 + n_p + n_d + len(plan)]
        (ixa, ixb, pya, pyb, acc,
         si0, si1, sp0, sp1) = refs[1 + n_p + n_d + len(plan):]
        cid = jax.lax.axis_index("c")
        tid = jax.lax.axis_index("s")

        for c in (0, 1):
            for (ri, g, r9) in per_core[c]:
                @pl.when(cid == c)
                def _round(ri=ri, g=g, r9=r9):
                    rel = plan[ri]
                    lo, fr = rel["ranges"][g]
                    zrows = (fr + 128) // 16
                    zbase = tid * zrows
                    pltpu.sync_copy(zz.at[pl.ds(0, zrows)],
                                    acc.at[pl.ds(zbase, zrows)])
                    plsc.subcore_barrier()
                    ne16 = rel["ne16"]
                    base = tid * ne16
                    p_ref = ps[ri * 3 + r9]
                    d_ref = ds[rel["dstp_base"] + g]
                    nc = ne16 // _SC_C
                    npairs = nc // 2
                    if npairs:
                        pltpu.async_copy(d_ref.at[pl.ds(base, _SC_C)],
                                         ixa, si0)
                        pltpu.async_copy(p_ref.at[pl.ds(base, _SC_C)],
                                         pya, sp0)

                        @pl.loop(0, npairs)
                        def _(ci):
                            o0 = base + (2 * ci) * _SC_C
                            o1 = o0 + _SC_C
                            pltpu.async_copy(d_ref.at[pl.ds(o1, _SC_C)],
                                             ixb, si1)
                            pltpu.async_copy(p_ref.at[pl.ds(o1, _SC_C)],
                                             pyb, sp1)
                            pltpu.make_async_copy(
                                d_ref.at[pl.ds(o0, _SC_C)], ixa, si0).wait()
                            pltpu.make_async_copy(
                                p_ref.at[pl.ds(o0, _SC_C)], pya, sp0).wait()
                            pltpu.sync_copy(pya, acc.at[ixa], add=True)

                            @pl.when(ci < npairs - 1)
                            def _():
                                pltpu.async_copy(
                                    d_ref.at[pl.ds(o1 + _SC_C, _SC_C)],
                                    ixa, si0)
                                pltpu.async_copy(
                                    p_ref.at[pl.ds(o1 + _SC_C, _SC_C)],
                                    pya, sp0)
                            pltpu.make_async_copy(
                                d_ref.at[pl.ds(o1, _SC_C)], ixb, si1).wait()
                            pltpu.make_async_copy(
                                p_ref.at[pl.ds(o1, _SC_C)], pyb, sp1).wait()
                            pltpu.sync_copy(pyb, acc.at[ixb], add=True)
                    if nc % 2:
                        ot = base + (nc - 1) * _SC_C
                        h_it = pltpu.async_copy(d_ref.at[pl.ds(ot, _SC_C)],
                                                ixa, si0)
                        h_pt = pltpu.async_copy(p_ref.at[pl.ds(ot, _SC_C)],
                                                pya, sp0)
                        h_it.wait()
                        h_pt.wait()
                        pltpu.sync_copy(pya, acc.at[ixa], add=True)
                    plsc.subcore_barrier()
                    frt = fr // 16
                    pltpu.sync_copy(
                        acc.at[pl.ds(tid * frt, frt)],
                        outs[ri].at[pl.ds(r9 * rel["ndf"] + lo + tid * frt, frt)])
                    plsc.subcore_barrier()

    zmax = max((fr + 128) // 16 for rel in plan for (_, fr) in rel["ranges"])
    zeros = jnp.zeros((zmax, _PW), jnp.float32)
    return k(zeros, *P_all, *dstp_list)


def _proj_body(x_ref, w_ref, b_ref, o_ref, *, k):
    y = (
        jnp.dot(x_ref[...], w_ref[...], preferred_element_type=jnp.float32, precision=jax.lax.Precision.HIGHEST)
        + b_ref[...]
    )
    for s in range(k):
        o_ref[s, :, :] = y[:, s * _HIDDEN:(s + 1) * _HIDDEN]


def _proj(x, W, b, blk=1000):
    N = x.shape[0]
    K = W.shape[1]
    k = K // _HIDDEN
    return pl.pallas_call(
        functools.partial(_proj_body, k=k),
        grid=(pl.cdiv(N, blk),),
        in_specs=[
            pl.BlockSpec((blk, _HIDDEN), lambda i: (i, 0)),
            pl.BlockSpec((_HIDDEN, K), lambda i: (0, 0)),
            pl.BlockSpec((1, K), lambda i: (0, 0)),
        ],
        out_specs=pl.BlockSpec((k, blk, _HIDDEN), lambda i: (0, i, 0)),
        out_shape=jax.ShapeDtypeStruct((k, N, _HIDDEN), jnp.float32),
    )(x, W, b.reshape(1, K))


def _edge_body(xj_ref, xi_ref, attbd_ref, exp_ref, *o_refs):
    xj = xj_ref[...]
    xi = xi_ref[...]
    s = xi + xj
    e = jnp.where(s > 0, s, 0.2 * s)
    alpha = jnp.dot(e, attbd_ref[...], preferred_element_type=jnp.float32, precision=jax.lax.Precision.HIGHEST)
    ex = jnp.exp(alpha)
    w = xj * jnp.dot(ex, exp_ref[...], preferred_element_type=jnp.float32, precision=jax.lax.Precision.HIGHEST)
    o_refs[0][...] = w[:, 0:48]
    o_refs[1][...] = w[:, 48:96]
    o_refs[2][...] = jnp.concatenate(
        [w[:, 96:128], ex, jnp.zeros((ex.shape[0], 8), jnp.float32)], axis=1)


_EBLK = 1000


def _edge_compute(G_src, G_dst, pos_s, pos_d, ne_pad, att):
    nb = pl.cdiv(ne_pad, _EBLK)
    assert pos_s % _EBLK == 0 and pos_d % _EBLK == 0
    pbs = pos_s // _EBLK
    pbd = pos_d // _EBLK
    attbd = att.reshape(-1)[:, None] * jnp.repeat(
        jnp.eye(_HEADS, dtype=jnp.float32), _HD, axis=0)
    expand = jnp.repeat(jnp.eye(_HEADS, dtype=jnp.float32), _HD, axis=1)
    return pl.pallas_call(
        _edge_body,
        grid=(nb,),
        in_specs=[
            pl.BlockSpec((_EBLK, _HIDDEN), lambda i: (pbs + i, 0)),
            pl.BlockSpec((_EBLK, _HIDDEN), lambda i: (pbd + i, 0)),
            pl.BlockSpec((_HIDDEN, _HEADS), lambda i: (0, 0)),
            pl.BlockSpec((_HEADS, _HIDDEN), lambda i: (0, 0)),
        ],
        out_specs=[pl.BlockSpec((_EBLK, _PW), lambda i: (i, 0))] * 3,
        out_shape=[jax.ShapeDtypeStruct((ne_pad, _PW), jnp.float32)] * 3,
    )(G_src, G_dst, attbd, expand)


def _final_body(x_ref, gam_ref, bet_ref, bias_ref, exp_ref, *o_refs):
    out_ref = o_refs[-1]
    agg = None
    for ri, o_ref in enumerate(o_refs[:-1]):
        ob = o_ref[...]
        num = jnp.concatenate([ob[0], ob[1], ob[2][:, 0:32]], axis=1)
        den = ob[2][:, 32:40]
        den_rep = jnp.dot(den, exp_ref[...],
                          preferred_element_type=jnp.float32, precision=jax.lax.Precision.HIGHEST) + 1e-16
        o = num / den_rep + bias_ref[ri:ri + 1, :]
        agg = o if agg is None else agg + o
    h = jnp.where(agg > 0, agg, jnp.exp(agg) - 1.0)
    y = x_ref[...] + h
    mu = jnp.mean(y, axis=-1, keepdims=True)
    var = jnp.mean((y - mu) ** 2, axis=-1, keepdims=True)
    out_ref[...] = (y - mu) / jnp.sqrt(var + 1e-5) * gam_ref[...] + bet_ref[...]


def _finalize(x, gamma, beta, biases, Os):
    N = x.shape[0]
    R = len(Os)
    blk = _EBLK
    expand = jnp.repeat(jnp.eye(_HEADS, dtype=jnp.float32), _HD, axis=1)
    return pl.pallas_call(
        _final_body,
        grid=(N // blk,),
        in_specs=[
            pl.BlockSpec((blk, _HIDDEN), lambda i: (i, 0)),
            pl.BlockSpec((1, _HIDDEN), lambda i: (0, 0)),
            pl.BlockSpec((1, _HIDDEN), lambda i: (0, 0)),
            pl.BlockSpec((R, _HIDDEN), lambda i: (0, 0)),
            pl.BlockSpec((_HEADS, _HIDDEN), lambda i: (0, 0)),
        ] + [pl.BlockSpec((3, blk, _PW), lambda i: (0, i, 0))] * R,
        out_specs=pl.BlockSpec((blk, _HIDDEN), lambda i: (i, 0)),
        out_shape=jax.ShapeDtypeStruct((N, _HIDDEN), jnp.float32),
    )(x, gamma.reshape(1, -1), beta.reshape(1, -1), jnp.stack(biases), expand,
      *Os)


def _head_body(b_ref, x_ref, gam_ref, bet_ref, o_ref, *, scale):
    y = jnp.dot(b_ref[...], x_ref[...],
                preferred_element_type=jnp.float32, precision=jax.lax.Precision.HIGHEST) * scale
    mu = jnp.mean(y, axis=-1, keepdims=True)
    var = jnp.mean((y - mu) ** 2, axis=-1, keepdims=True)
    o_ref[...] = (y - mu) / jnp.sqrt(var + 1e-5) * gam_ref[...] + bet_ref[...]


def _head(batch, x, gamma, beta):
    B, F = batch.shape
    return pl.pallas_call(
        functools.partial(_head_body, scale=1.0 / np.sqrt(F)),
        out_shape=jax.ShapeDtypeStruct((B, _HIDDEN), jnp.float32),
    )(batch, x, gamma.reshape(1, -1), beta.reshape(1, -1))


def kernel(batch_gene, batch_meth, batch_mirna, edge_index, params):
    n_nodes = {t: params["node_emb"][t].shape[0] for t in _NT}

    plan = []
    dstp_list = []
    for ri, (name, st, dt) in enumerate(_RELS):
        dst = edge_index[name][1]
        ne = dst.shape[0]
        ne_pad = _round_up(ne, _EPAD)
        ndf = _round_up(n_nodes[dt], 128)
        ranges = []
        lo = 0
        while lo < ndf:
            fr = min(_RS, ndf - lo)
            ranges.append((lo, fr))
            lo += fr
        dstp_base = len(dstp_list)
        for (lo, fr) in ranges:
            ok = (dst >= lo) & (dst < lo + fr)
            arr = jnp.where(ok, dst - lo, fr).astype(jnp.int32)
            dstp_list.append(jnp.concatenate(
                [arr, jnp.full((ne_pad - ne,), fr, jnp.int32)]))
        plan.append(dict(name=name, ne_pad=ne_pad, ne16=ne_pad // 16,
                         ndf=ndf, ranges=ranges, dstp_base=dstp_base))

    req = {t: 0 for t in _NT}
    pos_probe = {}
    for t in _NT:
        pos = 0
        for name in _SRC_RELS[t]:
            pos_probe[(name, "src")] = pos
            pos += edge_index[name][0].shape[0]
        for name in _DST_RELS[t]:
            pos_probe[(name, "dst")] = pos
            pos += edge_index[name][1].shape[0]
    for ri, (name, st, dt) in enumerate(_RELS):
        nb = -(-plan[ri]["ne_pad"] // _EBLK) * _EBLK
        req[st] = max(req[st], pos_probe[(name, "src")] + nb)
        req[dt] = max(req[dt], pos_probe[(name, "dst")] + nb)

    gidx, gslices = _build_gather_indices(edge_index, n_nodes, req)

    x = {t: params["node_emb"][t] for t in _NT}

    for l in range(_NL):
        conv = params["convs"][l]
        table = {}
        for t in _NT:
            Ws = [conv[n]["Wl"] for n in _SRC_RELS[t]] + [conv[n]["Wr"] for n in _DST_RELS[t]]
            bs = [conv[n]["bl"] for n in _SRC_RELS[t]] + [conv[n]["br"] for n in _DST_RELS[t]]
            table[t] = _proj(x[t], jnp.concatenate(Ws, axis=1), jnp.concatenate(bs, axis=0))

        G = {t: _sc_gather(table[t].reshape(-1, _HIDDEN), gidx[t]) for t in _NT}

        P_all = []
        for ri, (name, st, dt) in enumerate(_RELS):
            P = _edge_compute(G[st], G[dt], gslices[name]["src"],
                              gslices[name]["dst"], plan[ri]["ne_pad"],
                              conv[name]["att"])
            P_all.extend(P)
        Oraw = _sc_scatter_layer(plan, P_all, dstp_list)
        O = {name: Oraw[ri].reshape(3, plan[ri]["ndf"], _PW)
             for ri, (name, _, _) in enumerate(_RELS)}

        nxt = {}
        for t in _NT:
            Os = [O[n] for n in _DST_RELS[t]]
            biases = [conv[n]["bias"] for n in _DST_RELS[t]]
            ln = params["norms"][l][t]
            nxt[t] = _finalize(x[t], ln["gamma"], ln["beta"], biases, Os)
        x = nxt

    on = params["out_norm"]
    z_gene = _head(batch_gene, x["gene"], on["gene"]["gamma"], on["gene"]["beta"])
    z_cpg = _head(batch_meth, x["cpg"], on["cpg"]["gamma"], on["cpg"]["beta"])
    z_mirna = _head(batch_mirna, x["mirna"], on["mirna"]["gamma"], on["mirna"]["beta"])
    return (z_gene, z_cpg, z_mirna)


# speedup vs baseline: 1.5251x; 1.0032x over previous
import functools

import jax
import jax.numpy as jnp
import numpy as np
from jax.experimental import pallas as pl
from jax.experimental.pallas import tpu as pltpu
from jax.experimental.pallas import tpu_sc as plsc

_HIDDEN = 128
_HEADS = 8
_HD = _HIDDEN // _HEADS
_NL = 2
_RELS = [
    ("regulates", "cpg", "gene"),
    ("regulated_by", "gene", "cpg"),
    ("targets", "mirna", "gene"),
    ("targeted_by", "gene", "mirna"),
    ("interacts", "gene", "gene"),
    ("self_gene", "gene", "gene"),
    ("self_cpg", "cpg", "cpg"),
    ("self_mirna", "mirna", "mirna"),
]
_NT = ["gene", "cpg", "mirna"]
_SRC_RELS = {t: [n for (n, st, dt) in _RELS if st == t] for t in _NT}
_DST_RELS = {t: [n for (n, st, dt) in _RELS if dt == t] for t in _NT}

_SC_CORES = 2
_SC_TILES = 16
_NW = _SC_CORES * _SC_TILES
_GCHUNK = 96


def _sc_gather(table, idx):
    M = idx.shape[0]
    assert M % (_NW * _GCHUNK) == 0
    per_w = M // _NW
    n_chunks = per_w // _GCHUNK
    mesh = plsc.VectorSubcoreMesh(core_axis_name="c", subcore_axis_name="s")

    @functools.partial(
        pl.kernel,
        mesh=mesh,
        out_type=jax.ShapeDtypeStruct((M, _HIDDEN), jnp.float32),
        scratch_types=[
            pltpu.VMEM((_GCHUNK,), jnp.int32),
            pltpu.VMEM((_GCHUNK,), jnp.int32),
            pltpu.VMEM((_GCHUNK, _HIDDEN), jnp.float32),
            pltpu.VMEM((_GCHUNK, _HIDDEN), jnp.float32),
            pltpu.SemaphoreType.DMA,
            pltpu.SemaphoreType.DMA,
            pltpu.SemaphoreType.DMA,
            pltpu.SemaphoreType.DMA,
            pltpu.SemaphoreType.DMA,
            pltpu.SemaphoreType.DMA,
        ],
    )
    def k(table_hbm, idx_hbm, out_hbm, ia, ib, ra, rb,
          si0, si1, sg0, sg1, so0, so1):
        wid = jax.lax.axis_index("s") * _SC_CORES + jax.lax.axis_index("c")
        base = wid * per_w

        @pl.loop(0, n_chunks // 2)
        def _(g):
            o0 = base + (2 * g) * _GCHUNK
            o1 = o0 + _GCHUNK

            @pl.when(g > 0)
            def _():
                pltpu.make_async_copy(
                    ra, out_hbm.at[pl.ds(o0, _GCHUNK)], so0).wait()
                pltpu.make_async_copy(
                    rb, out_hbm.at[pl.ds(o1, _GCHUNK)], so1).wait()
            pltpu.sync_copy(idx_hbm.at[pl.ds(o0, _GCHUNK)], ia)
            h_g0 = pltpu.async_copy(table_hbm.at[ia], ra, sg0)
            pltpu.sync_copy(idx_hbm.at[pl.ds(o1, _GCHUNK)], ib)
            h_g1 = pltpu.async_copy(table_hbm.at[ib], rb, sg1)
            h_g0.wait()
            pltpu.async_copy(ra, out_hbm.at[pl.ds(o0, _GCHUNK)], so0)
            h_g1.wait()
            pltpu.async_copy(rb, out_hbm.at[pl.ds(o1, _GCHUNK)], so1)

        pltpu.make_async_copy(ra, out_hbm.at[pl.ds(base, _GCHUNK)], so0).wait()
        pltpu.make_async_copy(rb, out_hbm.at[pl.ds(base, _GCHUNK)], so1).wait()

    return k(table, idx)


def _build_gather_indices(edge_index, n_nodes, req):
    gidx = {}
    slices = {}
    for t in _NT:
        parts = []
        pos = 0
        for name in _SRC_RELS[t]:
            s = _SRC_RELS[t].index(name)
            src = edge_index[name][0]
            parts.append(src + s * n_nodes[t])
            slices.setdefault(name, {})["src"] = pos
            pos += src.shape[0]
        for name in _DST_RELS[t]:
            s = len(_SRC_RELS[t]) + _DST_RELS[t].index(name)
            dst = edge_index[name][1]
            parts.append(dst + s * n_nodes[t])
            slices.setdefault(name, {})["dst"] = pos
            pos += dst.shape[0]
        cat = jnp.concatenate(parts)
        need = max(pos, req.get(t, 0))
        tot = _round_up(need, _NW * _GCHUNK * 2)
        if tot > pos:
            cat = jnp.concatenate([cat, jnp.zeros((tot - pos,), jnp.int32)])
        gidx[t] = cat
    return gidx, slices


_RS = 24448
_ACC_ROWS = _RS + 128
_SC_C = 256
_PW = 48
_EPAD = 4096


def _round_up(v, m):
    return (v + m - 1) // m * m


def _sc_scatter_layer(plan, P_all, dstp_list):
    mesh = plsc.VectorSubcoreMesh(core_axis_name="c", subcore_axis_name="s")
    rounds = []
    for ri, rel in enumerate(plan):
        for g in range(len(rel["ranges"])):
            for r9 in range(3):
                rounds.append((ri, g, r9))
    def _cost(z):
        rel = plan[z[0]]
        fr = rel["ranges"][z[1]][1]
        return rel["ne16"] * 196 + 2 * ((fr + 128) // 16) * 192
    loads = [0, 0]
    per_core = ([], [])
    for rnd in sorted(rounds, key=lambda z: -_cost(z)):
        c = 0 if loads[0] <= loads[1] else 1
        per_core[c].append(rnd)
        loads[c] += _cost(rnd)

    n_p = len(P_all)
    n_d = len(dstp_list)
    out_types = [jax.ShapeDtypeStruct((3 * rel["ndf"], _PW), jnp.float32)
                 for rel in plan]

    @functools.partial(
        pl.kernel,
        mesh=mesh,
        out_type=out_types,
        compiler_params=pltpu.CompilerParams(use_tc_tiling_on_sc=False),
        scratch_types=[
            pltpu.VMEM((_SC_C,), jnp.int32),
            pltpu.VMEM((_SC_C,), jnp.int32),
            pltpu.VMEM((_SC_C, _PW), jnp.float32),
            pltpu.VMEM((_SC_C, _PW), jnp.float32),
            pltpu.VMEM_SHARED((_ACC_ROWS, _PW), jnp.float32),
            pltpu.SemaphoreType.DMA,
            pltpu.SemaphoreType.DMA,
            pltpu.SemaphoreType.DMA,
            pltpu.SemaphoreType.DMA,
        ],
    )
    def k(*refs):
        zz = refs[0]
        ps = refs[1:1 + n_p]
        ds = refs[1 + n_p:1 + n_p + n_d]
        outs = refs[1 + n_p + n_d:1 + n_p + n_d + len(plan)]
        (ixa, ixb, pya, pyb, acc,
         si0, si1, sp0, sp1) = refs[1 + n_p + n_d + len(plan):]
        cid = jax.lax.axis_index("c")
        tid = jax.lax.axis_index("s")

        for c in (0, 1):
            for (ri, g, r9) in per_core[c]:
                @pl.when(cid == c)
                def _round(ri=ri, g=g, r9=r9):
                    rel = plan[ri]
                    lo, fr = rel["ranges"][g]
                    zrows = (fr + 128) // 16
                    zbase = tid * zrows
                    pltpu.sync_copy(zz.at[pl.ds(0, zrows)],
                                    acc.at[pl.ds(zbase, zrows)])
                    plsc.subcore_barrier()
                    ne16 = rel["ne16"]
                    base = tid * ne16
                    p_ref = ps[ri * 3 + r9]
                    d_ref = ds[rel["dstp_base"] + g]
                    nc = ne16 // _SC_C
                    npairs = nc // 2
                    if npairs:
                        pltpu.async_copy(d_ref.at[pl.ds(base, _SC_C)],
                                         ixa, si0)
                        pltpu.async_copy(p_ref.at[pl.ds(base, _SC_C)],
                                         pya, sp0)

                        @pl.loop(0, npairs)
                        def _(ci):
                            o0 = base + (2 * ci) * _SC_C
                            o1 = o0 + _SC_C
                            pltpu.async_copy(d_ref.at[pl.ds(o1, _SC_C)],
                                             ixb, si1)
                            pltpu.async_copy(p_ref.at[pl.ds(o1, _SC_C)],
                                             pyb, sp1)
                            pltpu.make_async_copy(
                                d_ref.at[pl.ds(o0, _SC_C)], ixa, si0).wait()
                            pltpu.make_async_copy(
                                p_ref.at[pl.ds(o0, _SC_C)], pya, sp0).wait()
                            pltpu.sync_copy(pya, acc.at[ixa], add=True)

                            @pl.when(ci < npairs - 1)
                            def _():
                                pltpu.async_copy(
                                    d_ref.at[pl.ds(o1 + _SC_C, _SC_C)],
                                    ixa, si0)
                                pltpu.async_copy(
                                    p_ref.at[pl.ds(o1 + _SC_C, _SC_C)],
                                    pya, sp0)
                            pltpu.make_async_copy(
                                d_ref.at[pl.ds(o1, _SC_C)], ixb, si1).wait()
                            pltpu.make_async_copy(
                                p_ref.at[pl.ds(o1, _SC_C)], pyb, sp1).wait()
                            pltpu.sync_copy(pyb, acc.at[ixb], add=True)
                    if nc % 2:
                        ot = base + (nc - 1) * _SC_C
                        h_it = pltpu.async_copy(d_ref.at[pl.ds(ot, _SC_C)],
                                                ixa, si0)
                        h_pt = pltpu.async_copy(p_ref.at[pl.ds(ot, _SC_C)],
                                                pya, sp0)
                        h_it.wait()
                        h_pt.wait()
                        pltpu.sync_copy(pya, acc.at[ixa], add=True)
                    plsc.subcore_barrier()
                    frt = fr // 16
                    pltpu.sync_copy(
                        acc.at[pl.ds(tid * frt, frt)],
                        outs[ri].at[pl.ds(r9 * rel["ndf"] + lo + tid * frt, frt)])
                    plsc.subcore_barrier()

    zmax = max((fr + 128) // 16 for rel in plan for (_, fr) in rel["ranges"])
    zeros = jnp.zeros((zmax, _PW), jnp.float32)
    return k(zeros, *P_all, *dstp_list)


def _proj_body(x_ref, w_ref, b_ref, o_ref, *, k):
    y = (
        jnp.dot(x_ref[...], w_ref[...], preferred_element_type=jnp.float32, precision=jax.lax.Precision.HIGHEST)
        + b_ref[...]
    )
    for s in range(k):
        o_ref[s, :, :] = y[:, s * _HIDDEN:(s + 1) * _HIDDEN]


def _proj(x, W, b, blk=1000):
    N = x.shape[0]
    K = W.shape[1]
    k = K // _HIDDEN
    return pl.pallas_call(
        functools.partial(_proj_body, k=k),
        grid=(pl.cdiv(N, blk),),
        in_specs=[
            pl.BlockSpec((blk, _HIDDEN), lambda i: (i, 0)),
            pl.BlockSpec((_HIDDEN, K), lambda i: (0, 0)),
            pl.BlockSpec((1, K), lambda i: (0, 0)),
        ],
        out_specs=pl.BlockSpec((k, blk, _HIDDEN), lambda i: (0, i, 0)),
        out_shape=jax.ShapeDtypeStruct((k, N, _HIDDEN), jnp.float32),
    )(x, W, b.reshape(1, K))


def _edge_body(xj_ref, xi_ref, attbd_ref, exp_ref, *o_refs):
    xj = xj_ref[...]
    xi = xi_ref[...]
    s = xi + xj
    e = jnp.where(s > 0, s, 0.2 * s)
    alpha = jnp.dot(e, attbd_ref[...], preferred_element_type=jnp.float32, precision=jax.lax.Precision.HIGHEST)
    ex = jnp.exp(alpha)
    w = xj * jnp.dot(ex, exp_ref[...], preferred_element_type=jnp.float32, precision=jax.lax.Precision.HIGHEST)
    o_refs[0][...] = w[:, 0:48]
    o_refs[1][...] = w[:, 48:96]
    o_refs[2][...] = jnp.concatenate(
        [w[:, 96:128], ex, jnp.zeros((ex.shape[0], 8), jnp.float32)], axis=1)


_EBLK = 1000


def _edge_compute(G_src, G_dst, pos_s, pos_d, ne_pad, att):
    nb = pl.cdiv(ne_pad, _EBLK)
    assert pos_s % _EBLK == 0 and pos_d % _EBLK == 0
    pbs = pos_s // _EBLK
    pbd = pos_d // _EBLK
    attbd = att.reshape(-1)[:, None] * jnp.repeat(
        jnp.eye(_HEADS, dtype=jnp.float32), _HD, axis=0)
    expand = jnp.repeat(jnp.eye(_HEADS, dtype=jnp.float32), _HD, axis=1)
    return pl.pallas_call(
        _edge_body,
        grid=(nb,),
        in_specs=[
            pl.BlockSpec((_EBLK, _HIDDEN), lambda i: (pbs + i, 0)),
            pl.BlockSpec((_EBLK, _HIDDEN), lambda i: (pbd + i, 0)),
            pl.BlockSpec((_HIDDEN, _HEADS), lambda i: (0, 0)),
            pl.BlockSpec((_HEADS, _HIDDEN), lambda i: (0, 0)),
        ],
        out_specs=[pl.BlockSpec((_EBLK, _PW), lambda i: (i, 0))] * 3,
        out_shape=[jax.ShapeDtypeStruct((ne_pad, _PW), jnp.float32)] * 3,
    )(G_src, G_dst, attbd, expand)


def _final_body(x_ref, gam_ref, bet_ref, bias_ref, exp_ref, *o_refs):
    out_ref = o_refs[-1]
    agg = None
    for ri, o_ref in enumerate(o_refs[:-1]):
        ob = o_ref[...]
        num = jnp.concatenate([ob[0], ob[1], ob[2][:, 0:32]], axis=1)
        den = ob[2][:, 32:40]
        den_rep = jnp.dot(den, exp_ref[...],
                          preferred_element_type=jnp.float32, precision=jax.lax.Precision.HIGHEST) + 1e-16
        o = num / den_rep + bias_ref[ri:ri + 1, :]
        agg = o if agg is None else agg + o
    h = jnp.where(agg > 0, agg, jnp.exp(agg) - 1.0)
    y = x_ref[...] + h
    mu = jnp.mean(y, axis=-1, keepdims=True)
    var = jnp.mean((y - mu) ** 2, axis=-1, keepdims=True)
    out_ref[...] = (y - mu) / jnp.sqrt(var + 1e-5) * gam_ref[...] + bet_ref[...]


def _finalize(x, gamma, beta, biases, Os):
    N = x.shape[0]
    R = len(Os)
    blk = _EBLK
    expand = jnp.repeat(jnp.eye(_HEADS, dtype=jnp.float32), _HD, axis=1)
    return pl.pallas_call(
        _final_body,
        grid=(N // blk,),
        in_specs=[
            pl.BlockSpec((blk, _HIDDEN), lambda i: (i, 0)),
            pl.BlockSpec((1, _HIDDEN), lambda i: (0, 0)),
            pl.BlockSpec((1, _HIDDEN), lambda i: (0, 0)),
            pl.BlockSpec((R, _HIDDEN), lambda i: (0, 0)),
            pl.BlockSpec((_HEADS, _HIDDEN), lambda i: (0, 0)),
        ] + [pl.BlockSpec((3, blk, _PW), lambda i: (0, i, 0))] * R,
        out_specs=pl.BlockSpec((blk, _HIDDEN), lambda i: (i, 0)),
        out_shape=jax.ShapeDtypeStruct((N, _HIDDEN), jnp.float32),
    )(x, gamma.reshape(1, -1), beta.reshape(1, -1), jnp.stack(biases), expand,
      *Os)


def _head_body(b_ref, x_ref, gam_ref, bet_ref, o_ref, *, scale):
    y = jnp.dot(b_ref[...], x_ref[...],
                preferred_element_type=jnp.float32, precision=jax.lax.Precision.HIGHEST) * scale
    mu = jnp.mean(y, axis=-1, keepdims=True)
    var = jnp.mean((y - mu) ** 2, axis=-1, keepdims=True)
    o_ref[...] = (y - mu) / jnp.sqrt(var + 1e-5) * gam_ref[...] + bet_ref[...]


def _head(batch, x, gamma, beta):
    B, F = batch.shape
    return pl.pallas_call(
        functools.partial(_head_body, scale=1.0 / np.sqrt(F)),
        out_shape=jax.ShapeDtypeStruct((B, _HIDDEN), jnp.float32),
    )(batch, x, gamma.reshape(1, -1), beta.reshape(1, -1))


def kernel(batch_gene, batch_meth, batch_mirna, edge_index, params):
    n_nodes = {t: params["node_emb"][t].shape[0] for t in _NT}

    plan = []
    dstp_list = []
    for ri, (name, st, dt) in enumerate(_RELS):
        dst = edge_index[name][1]
        ne = dst.shape[0]
        ne_pad = _round_up(ne, _EPAD)
        ndf = _round_up(n_nodes[dt], 128)
        ranges = []
        lo = 0
        while lo < ndf:
            fr = min(_RS, ndf - lo)
            ranges.append((lo, fr))
            lo += fr
        dstp_base = len(dstp_list)
        for (lo, fr) in ranges:
            ok = (dst >= lo) & (dst < lo + fr)
            arr = jnp.where(ok, dst - lo, fr).astype(jnp.int32)
            dstp_list.append(jnp.concatenate(
                [arr, jnp.full((ne_pad - ne,), fr, jnp.int32)]))
        plan.append(dict(name=name, ne_pad=ne_pad, ne16=ne_pad // 16,
                         ndf=ndf, ranges=ranges, dstp_base=dstp_base))

    req = {t: 0 for t in _NT}
    pos_probe = {}
    for t in _NT:
        pos = 0
        for name in _SRC_RELS[t]:
            pos_probe[(name, "src")] = pos
            pos += edge_index[name][0].shape[0]
        for name in _DST_RELS[t]:
            pos_probe[(name, "dst")] = pos
            pos += edge_index[name][1].shape[0]
    for ri, (name, st, dt) in enumerate(_RELS):
        nb = -(-plan[ri]["ne_pad"] // _EBLK) * _EBLK
        req[st] = max(req[st], pos_probe[(name, "src")] + nb)
        req[dt] = max(req[dt], pos_probe[(name, "dst")] + nb)

    gidx, gslices = _build_gather_indices(edge_index, n_nodes, req)

    x = {t: params["node_emb"][t] for t in _NT}

    for l in range(_NL):
        conv = params["convs"][l]
        table = {}
        for t in _NT:
            Ws = [conv[n]["Wl"] for n in _SRC_RELS[t]] + [conv[n]["Wr"] for n in _DST_RELS[t]]
            bs = [conv[n]["bl"] for n in _SRC_RELS[t]] + [conv[n]["br"] for n in _DST_RELS[t]]
            table[t] = _proj(x[t], jnp.concatenate(Ws, axis=1), jnp.concatenate(bs, axis=0))

        G = {t: _sc_gather(table[t].reshape(-1, _HIDDEN), gidx[t]) for t in _NT}

        P_all = []
        for ri, (name, st, dt) in enumerate(_RELS):
            P = _edge_compute(G[st], G[dt], gslices[name]["src"],
                              gslices[name]["dst"], plan[ri]["ne_pad"],
                              conv[name]["att"])
            P_all.extend(P)
        Oraw = _sc_scatter_layer(plan, P_all, dstp_list)
        O = {name: Oraw[ri].reshape(3, plan[ri]["ndf"], _PW)
             for ri, (name, _, _) in enumerate(_RELS)}

        nxt = {}
        for t in _NT:
            Os = [O[n] for n in _DST_RELS[t]]
            biases = [conv[n]["bias"] for n in _DST_RELS[t]]
            ln = params["norms"][l][t]
            nxt[t] = _finalize(x[t], ln["gamma"], ln["beta"], biases, Os)
        x = nxt

    on = params["out_norm"]
    z_gene = _head(batch_gene, x["gene"], on["gene"]["gamma"], on["gene"]["beta"])
    z_cpg = _head(batch_meth, x["cpg"], on["cpg"]["gamma"], on["cpg"]["beta"])
    z_mirna = _head(batch_mirna, x["mirna"], on["mirna"]["gamma"], on["mirna"]["beta"])
    return (z_gene, z_cpg, z_mirna)
